# Initial kernel scaffold; baseline (speedup 1.0000x reference)
import jax
import jax.numpy as jnp
from jax.experimental import pallas as pl


def kernel(memory, controls, read_weights):
    raise NotImplementedError("write your pallas kernel here")


# speedup vs baseline: 3.3139x; 3.3139x over previous
import jax
import jax.numpy as jnp
from jax import lax
from jax.experimental import pallas as pl

EPS = 1e-08
_LOG_EPS = float(jnp.log(jnp.float32(EPS)))


def _write_head_kernel(mem_ref, ctrl_ref, out_ref):
    mem = mem_ref[0]
    ctrl = ctrl_ref[0]
    c1, c2, w = mem.shape

    keys = ctrl[:, 0:w]
    erase = jax.nn.sigmoid(ctrl[:, w:2 * w])
    add = ctrl[:, 2 * w:3 * w]
    beta = jax.nn.softplus(ctrl[:, -3:-2])
    alloc_gate = jax.nn.sigmoid(ctrl[:, -2:-1])
    write_gate = jax.nn.sigmoid(ctrl[:, -1:])

    keys3 = keys[:, None, :]
    dot = jnp.sum(mem * keys3, axis=-1)
    nrm2 = jnp.sum(mem * mem, axis=-1)
    key_norm = jnp.sqrt(jnp.sum(keys * keys))
    scores = dot / (key_norm * jnp.sqrt(nrm2) + EPS) * beta

    smax = jnp.max(scores)
    e = jnp.exp(scores - smax)
    content = e / jnp.sum(e)

    idx = (lax.broadcasted_iota(jnp.int32, (c1, c2), 0) * c2
           + lax.broadcasted_iota(jnp.int32, (c1, c2), 1)).astype(jnp.float32)
    alloc = (1.0 - EPS) * jnp.exp(idx * _LOG_EPS)

    ww = write_gate * (alloc_gate * alloc + (1.0 - alloc_gate) * content)
    wts = ww[:, :, None]
    out_ref[0] = mem * (1.0 - wts * erase[:, None, :]) + wts * add[:, None, :]


def kernel(memory, controls, read_weights):
    b, c, w = memory.shape
    c1 = 128
    c2 = c // c1
    n = controls.shape[-1]
    mem4 = memory.reshape(b, c1, c2, w)
    ctrl3 = controls.reshape(b, 1, n)
    out = pl.pallas_call(
        _write_head_kernel,
        grid=(b,),
        in_specs=[
            pl.BlockSpec((1, c1, c2, w), lambda i: (i, 0, 0, 0)),
            pl.BlockSpec((1, 1, n), lambda i: (i, 0, 0)),
        ],
        out_specs=pl.BlockSpec((1, c1, c2, w), lambda i: (i, 0, 0, 0)),
        out_shape=jax.ShapeDtypeStruct((b, c1, c2, w), memory.dtype),
    )(mem4, ctrl3)
    return out.reshape(b, c, w)
